# Initial kernel scaffold; baseline (speedup 1.0000x reference)
import jax
import jax.numpy as jnp
from jax.experimental import pallas as pl


def kernel(x, edge_index, params):
    raise NotImplementedError("write your pallas kernel here")


# speedup vs baseline: 7.5238x; 7.5238x over previous
import functools

import jax
import jax.numpy as jnp
from jax import lax
from jax.experimental import pallas as pl
from jax.experimental.pallas import tpu as pltpu
from jax.experimental.pallas import tpu_sc as plsc

_NC = 2
_NS = 16
_LANES = 16

_CHUNK = 128
_GROUP = 10
_RB = 4
_ZROWS = 56


def _sc_agg_body(hs2, src1d, dst1d, out, acc, isrc0, isrc1, idst0, idst1,
                 rows0, rows1, rows2, rows3, zbuf, isem, gsem,
                 n_nodes, hh, n_groups):
  c = lax.axis_index("c")
  s = lax.axis_index("s")
  cbase = c * n_nodes

  isrc = (isrc0, isrc1)
  idst = (idst0, idst1)
  rows = (rows0, rows1, rows2, rows3)

  stripe = -(-n_nodes // _NS)
  stripe += (-stripe) % _ZROWS

  def _zero_zbuf(r, carry):
    zbuf[r, pl.ds(0, _LANES)] = jnp.zeros((_LANES,), jnp.float32)
    zbuf[r, pl.ds(_LANES, _LANES)] = jnp.zeros((_LANES,), jnp.float32)
    return carry
  lax.fori_loop(0, _ZROWS, _zero_zbuf, 0)

  def _zero_acc(j, carry):
    pltpu.sync_copy(zbuf, acc.at[pl.ds(s * stripe + j * _ZROWS, _ZROWS)])
    return carry
  lax.fori_loop(0, stripe // _ZROWS, _zero_acc, 0)
  plsc.subcore_barrier()

  base_g = n_groups // _NS
  extra = n_groups % _NS
  nloc = base_g + jnp.where(s < extra, 1, 0)
  g0 = s * base_g + jnp.minimum(s, extra)

  def _idx_start(p, gi):
    base = (g0 + gi) * _GROUP * _CHUNK
    pltpu.async_copy(src1d.at[pl.ds(base, _GROUP * _CHUNK)], isrc[p],
                     isem.at[p])
    for k in range(_GROUP):
      pltpu.async_copy(dst1d.at[pl.ds(base + k * _CHUNK, _CHUNK)],
                       idst[p].at[k], isem.at[p])

  def _idx_wait_adjust(p, gi):
    base = (g0 + gi) * _GROUP * _CHUNK
    pltpu.make_async_copy(src1d.at[pl.ds(base, _GROUP * _CHUNK)], isrc[p],
                          isem.at[p]).wait()
    for k in range(_GROUP):
      pltpu.make_async_copy(dst1d.at[pl.ds(base + k * _CHUNK, _CHUNK)],
                            idst[p].at[k], isem.at[p]).wait()
    for m in range(_GROUP * _CHUNK // _LANES):
      sl = pl.ds(m * _LANES, _LANES)
      isrc[p][sl] = isrc[p][sl] + cbase

  def _gather_slice(p, k):
    return isrc[p].at[pl.ds(k * _CHUNK, _CHUNK)]

  def _fire(p, k, r):
    pltpu.async_copy(hs2.at[_gather_slice(p, k)], rows[r], gsem.at[r])

  def _drain_scatter(p, k, r):
    pltpu.make_async_copy(hs2.at[_gather_slice(p, k)], rows[r],
                          gsem.at[r]).wait()
    pltpu.sync_copy(rows[r], acc.at[idst[p].at[k]], add=True)

  @pl.when(nloc > 0)
  def _prologue():
    _idx_start(0, 0)

  def _steady(jj, carry):
    for p in (0, 1):
      gi = 2 * jj + p
      @pl.when(gi < nloc)
      def _():
        _idx_wait_adjust(p, gi)
        @pl.when(gi + 1 < nloc)
        def _():
          _idx_start(1 - p, gi + 1)
        base_r = (2 * p) % _RB
        _fire(p, 0, base_r)
        _fire(p, 1, (base_r + 1) % _RB)
        for k in range(_GROUP):
          if k + 2 < _GROUP:
            _fire(p, k + 2, (base_r + k + 2) % _RB)
          _drain_scatter(p, k, (base_r + k) % _RB)
    return carry
  lax.fori_loop(0, (nloc + 1) // 2, _steady, 0)

  plsc.subcore_barrier()

  tail = n_nodes - (_NS - 1) * stripe
  @pl.when(s < _NS - 1)
  def _wb_full():
    pltpu.sync_copy(acc.at[pl.ds(s * stripe, stripe)],
                    out.at[pl.ds(cbase + s * stripe, stripe)])
  @pl.when(s == _NS - 1)
  def _wb_tail():
    pltpu.sync_copy(acc.at[pl.ds((_NS - 1) * stripe, tail)],
                    out.at[pl.ds(cbase + (_NS - 1) * stripe, tail)])


def _make_sc_agg(n_nodes, n_edges, hh):
  assert n_edges % (_CHUNK * _GROUP) == 0
  stripe = -(-n_nodes // _NS)
  stripe += (-stripe) % _ZROWS
  assert stripe % _ZROWS == 0
  assert n_nodes - (_NS - 1) * stripe > 0
  assert (n_nodes - (_NS - 1) * stripe) % 8 == 0
  n_groups = n_edges // (_CHUNK * _GROUP)
  mesh = plsc.VectorSubcoreMesh(core_axis_name="c", subcore_axis_name="s")
  body = functools.partial(_sc_agg_body, n_nodes=n_nodes, hh=hh,
                           n_groups=n_groups)
  return pl.kernel(
      body,
      out_type=jax.ShapeDtypeStruct((_NC * n_nodes, hh), jnp.float32),
      mesh=mesh,
      scratch_types=[
          pltpu.VMEM_SHARED((_NS * stripe, hh), jnp.float32),
          pltpu.VMEM((_GROUP * _CHUNK,), jnp.int32),
          pltpu.VMEM((_GROUP * _CHUNK,), jnp.int32),
          pltpu.VMEM((_GROUP, _CHUNK), jnp.int32),
          pltpu.VMEM((_GROUP, _CHUNK), jnp.int32),
          pltpu.VMEM((_CHUNK, hh), jnp.float32),
          pltpu.VMEM((_CHUNK, hh), jnp.float32),
          pltpu.VMEM((_CHUNK, hh), jnp.float32),
          pltpu.VMEM((_CHUNK, hh), jnp.float32),
          pltpu.VMEM((_ZROWS, hh), jnp.float32),
          pltpu.SemaphoreType.DMA((2,)),
          pltpu.SemaphoreType.DMA((_RB,)),
      ],
      compiler_params=pltpu.CompilerParams(use_tc_tiling_on_sc=False),
      name="sc_gin_agg",
  )


_BLK = 1000


def _split_body(x_ref, out_ref, *, hh):
  x = x_ref[...]
  out_ref[0, :, :] = x[:, :hh]
  out_ref[1, :, :] = x[:, hh:]


def _mlp_body(eps_ref, h0_ref, h1_ref, a0_ref, a1_ref, w1_ref, b1_ref,
              w2_ref, b2_ref, z_ref, st_ref):
  i = pl.program_id(0)
  h = jnp.concatenate([h0_ref[...], h1_ref[...]], axis=1)
  a = jnp.concatenate([a0_ref[...], a1_ref[...]], axis=1)
  zin = h * eps_ref[0] + a
  dn = (((1,), (0,)), ((), ()))
  z1 = jnp.maximum(
      lax.dot_general(zin, w1_ref[...], dn,
                      preferred_element_type=jnp.float32) + b1_ref[...], 0.0)
  z2 = jnp.maximum(
      lax.dot_general(z1, w2_ref[...], dn,
                      preferred_element_type=jnp.float32) + b2_ref[...], 0.0)
  z_ref[...] = z2
  part = jnp.concatenate([jnp.sum(z2, axis=0, keepdims=True),
                          jnp.sum(z2 * z2, axis=0, keepdims=True)], axis=0)
  @pl.when(i == 0)
  def _():
    st_ref[...] = part
  @pl.when(i != 0)
  def _():
    st_ref[...] = st_ref[...] + part


def _bn_body(z_ref, st_ref, g_ref, be_ref, out_ref, *, n_nodes, hh):
  inv_n = 1.0 / n_nodes
  mean = st_ref[0:1, :] * inv_n
  var = st_ref[1:2, :] * inv_n - mean * mean
  scale = g_ref[...] * lax.rsqrt(var + 1e-5)
  shift = be_ref[...] - mean * scale
  y = z_ref[...] * scale + shift
  out_ref[0, :, :] = y[:, :hh]
  out_ref[1, :, :] = y[:, hh:]


def _head_body(b2_ref, h0_ref, h1_ref, w1_ref, b1_ref, w2_ref, out_ref):
  h = jnp.concatenate([h0_ref[...], h1_ref[...]], axis=1)
  dn = (((1,), (0,)), ((), ()))
  t = jnp.maximum(
      lax.dot_general(h, w1_ref[...], dn,
                      preferred_element_type=jnp.float32) + b1_ref[...], 0.0)
  o = jnp.sum(t * w2_ref[...], axis=1, keepdims=True)
  out_ref[...] = o + b2_ref[0]


def kernel(x, edge_index, params):
  n, hdim = x.shape
  hh = hdim // 2
  e = edge_index.shape[1]
  nb = n // _BLK
  assert n % _BLK == 0

  sc_agg = _make_sc_agg(n, e, hh)
  src1d = edge_index[0]
  dst1d = edge_index[1]

  def row_spec(index_map):
    return pl.BlockSpec((_BLK, hh), index_map)

  half_specs = [row_spec(lambda i: (i, 0)),
                row_spec(lambda i, _nbh=nb: (_nbh + i, 0))]
  full_spec = pl.BlockSpec((_BLK, hdim), lambda i: (i, 0))
  w_spec = pl.BlockSpec((hdim, hdim), lambda i: (0, 0))
  b_spec = pl.BlockSpec((1, hdim), lambda i: (0, 0))
  st_spec = pl.BlockSpec((2, hdim), lambda i: (0, 0))
  split_out_spec = pl.BlockSpec((2, _BLK, hh), lambda i: (0, i, 0))
  smem_spec = pl.BlockSpec(memory_space=pltpu.SMEM)

  split = pl.pallas_call(
      functools.partial(_split_body, hh=hh),
      grid=(nb,),
      in_specs=[full_spec],
      out_specs=split_out_spec,
      out_shape=jax.ShapeDtypeStruct((2, n, hh), jnp.float32),
  )

  mlp = pl.pallas_call(
      _mlp_body,
      grid=(nb,),
      in_specs=[smem_spec] + half_specs + half_specs
      + [w_spec, b_spec, w_spec, b_spec],
      out_specs=[full_spec, st_spec],
      out_shape=[jax.ShapeDtypeStruct((n, hdim), jnp.float32),
                 jax.ShapeDtypeStruct((2, hdim), jnp.float32)],
  )

  bn = pl.pallas_call(
      functools.partial(_bn_body, n_nodes=n, hh=hh),
      grid=(nb,),
      in_specs=[full_spec, st_spec, b_spec, b_spec],
      out_specs=split_out_spec,
      out_shape=jax.ShapeDtypeStruct((2, n, hh), jnp.float32),
  )

  head = pl.pallas_call(
      _head_body,
      grid=(nb,),
      in_specs=[smem_spec] + half_specs + [w_spec, b_spec, b_spec],
      out_specs=pl.BlockSpec((_BLK, 1), lambda i: (i, 0)),
      out_shape=jax.ShapeDtypeStruct((n, 1), jnp.float32),
  )

  hs2 = split(x).reshape(_NC * n, hh)
  for i in range(3):
    agg2 = sc_agg(hs2, src1d, dst1d)
    epl = (1.0 + params[f"eps{i}"]).reshape(1)
    z, st = mlp(epl, hs2, hs2, agg2, agg2,
                params[f"W1_{i}"].T, params[f"b1_{i}"].reshape(1, hdim),
                params[f"W2_{i}"].T, params[f"b2_{i}"].reshape(1, hdim))
    hs2 = bn(z, st, params[f"g{i}"].reshape(1, hdim),
             params[f"be{i}"].reshape(1, hdim)).reshape(_NC * n, hh)

  return head(params["lin2_b"], hs2, hs2,
              params["lin1_W"].T, params["lin1_b"].reshape(1, hdim),
              params["lin2_W"])


# speedup vs baseline: 10.9562x; 1.4562x over previous
import functools

import jax
import jax.numpy as jnp
from jax import lax
from jax.experimental import pallas as pl
from jax.experimental.pallas import tpu as pltpu
from jax.experimental.pallas import tpu_sc as plsc

_NC = 2
_NS = 16
_LANES = 16

_CHUNK = 128
_GROUP = 10
_RB = 4
_ZROWS = 56


def _sc_agg_body(hsi, eflat, out, acc, isrc0, isrc1, idst0, idst1,
                 rows0, rows1, rows2, rows3, zbuf, isem, gsem,
                 n_nodes, n_edges, hh, n_groups):
  c = lax.axis_index("c")
  s = lax.axis_index("s")

  isrc = (isrc0, isrc1)
  idst = (idst0, idst1)
  rows = (rows0, rows1, rows2, rows3)

  stripe = -(-n_nodes // _NS)
  stripe += (-stripe) % _ZROWS

  def _zero_zbuf(r, carry):
    zbuf[r, pl.ds(0, _LANES)] = jnp.zeros((_LANES,), jnp.float32)
    zbuf[r, pl.ds(_LANES, _LANES)] = jnp.zeros((_LANES,), jnp.float32)
    return carry
  lax.fori_loop(0, _ZROWS, _zero_zbuf, 0)

  def _zero_acc(j, carry):
    pltpu.sync_copy(zbuf, acc.at[pl.ds(s * stripe + j * _ZROWS, _ZROWS)])
    return carry
  lax.fori_loop(0, stripe // _ZROWS, _zero_acc, 0)
  plsc.subcore_barrier()

  base_g = n_groups // _NS
  extra = n_groups % _NS
  nloc = base_g + jnp.where(s < extra, 1, 0)
  g0 = s * base_g + jnp.minimum(s, extra)

  def _idx_start(p, gi):
    base = (g0 + gi) * _GROUP * _CHUNK
    pltpu.async_copy(eflat.at[pl.ds(base, _GROUP * _CHUNK)], isrc[p],
                     isem.at[p])
    for k in range(_GROUP):
      pltpu.async_copy(eflat.at[pl.ds(n_edges + base + k * _CHUNK, _CHUNK)],
                       idst[p].at[k], isem.at[p])

  def _idx_wait_adjust(p, gi):
    base = (g0 + gi) * _GROUP * _CHUNK
    pltpu.make_async_copy(eflat.at[pl.ds(base, _GROUP * _CHUNK)], isrc[p],
                          isem.at[p]).wait()
    for k in range(_GROUP):
      pltpu.make_async_copy(
          eflat.at[pl.ds(n_edges + base + k * _CHUNK, _CHUNK)],
          idst[p].at[k], isem.at[p]).wait()
    for m in range(_GROUP * _CHUNK // _LANES):
      sl = pl.ds(m * _LANES, _LANES)
      isrc[p][sl] = isrc[p][sl] * 2 + c

  def _gather_slice(p, k):
    return isrc[p].at[pl.ds(k * _CHUNK, _CHUNK)]

  def _fire(p, k, r):
    pltpu.async_copy(hsi.at[_gather_slice(p, k)], rows[r], gsem.at[r])

  def _drain_scatter(p, k, r):
    pltpu.make_async_copy(hsi.at[_gather_slice(p, k)], rows[r],
                          gsem.at[r]).wait()
    pltpu.sync_copy(rows[r], acc.at[idst[p].at[k]], add=True)

  @pl.when(nloc > 0)
  def _prologue():
    _idx_start(0, 0)

  def _steady(jj, carry):
    for p in (0, 1):
      gi = 2 * jj + p
      @pl.when(gi < nloc)
      def _():
        _idx_wait_adjust(p, gi)
        @pl.when(gi + 1 < nloc)
        def _():
          _idx_start(1 - p, gi + 1)
        base_r = (2 * p) % _RB
        _fire(p, 0, base_r)
        _fire(p, 1, (base_r + 1) % _RB)
        for k in range(_GROUP):
          if k + 2 < _GROUP:
            _fire(p, k + 2, (base_r + k + 2) % _RB)
          _drain_scatter(p, k, (base_r + k) % _RB)
    return carry
  lax.fori_loop(0, (nloc + 1) // 2, _steady, 0)

  plsc.subcore_barrier()

  tail = n_nodes - (_NS - 1) * stripe
  @pl.when(s < _NS - 1)
  def _wb_full():
    pltpu.sync_copy(acc.at[pl.ds(s * stripe, stripe)],
                    out.at[pl.ds(s * stripe, stripe), c])
  @pl.when(s == _NS - 1)
  def _wb_tail():
    pltpu.sync_copy(acc.at[pl.ds((_NS - 1) * stripe, tail)],
                    out.at[pl.ds((_NS - 1) * stripe, tail), c])


def _make_sc_agg(n_nodes, n_edges, hh):
  assert n_edges % (_CHUNK * _GROUP) == 0
  stripe = -(-n_nodes // _NS)
  stripe += (-stripe) % _ZROWS
  assert n_nodes - (_NS - 1) * stripe > 0
  assert (n_nodes - (_NS - 1) * stripe) % 8 == 0
  n_groups = n_edges // (_CHUNK * _GROUP)
  mesh = plsc.VectorSubcoreMesh(core_axis_name="c", subcore_axis_name="s")
  body = functools.partial(_sc_agg_body, n_nodes=n_nodes, n_edges=n_edges,
                           hh=hh, n_groups=n_groups)
  return pl.kernel(
      body,
      out_type=jax.ShapeDtypeStruct((n_nodes, _NC, hh), jnp.float32),
      mesh=mesh,
      scratch_types=[
          pltpu.VMEM_SHARED((_NS * stripe, hh), jnp.float32),
          pltpu.VMEM((_GROUP * _CHUNK,), jnp.int32),
          pltpu.VMEM((_GROUP * _CHUNK,), jnp.int32),
          pltpu.VMEM((_GROUP, _CHUNK), jnp.int32),
          pltpu.VMEM((_GROUP, _CHUNK), jnp.int32),
          pltpu.VMEM((_CHUNK, hh), jnp.float32),
          pltpu.VMEM((_CHUNK, hh), jnp.float32),
          pltpu.VMEM((_CHUNK, hh), jnp.float32),
          pltpu.VMEM((_CHUNK, hh), jnp.float32),
          pltpu.VMEM((_ZROWS, hh), jnp.float32),
          pltpu.SemaphoreType.DMA((2,)),
          pltpu.SemaphoreType.DMA((_RB,)),
      ],
      compiler_params=pltpu.CompilerParams(use_tc_tiling_on_sc=False),
      name="sc_gin_agg",
  )


_BLK = 1000


def _mlp_body(eps_ref, h_ref, a_ref, w1_ref, b1_ref, w2_ref, b2_ref,
              z_ref, st_ref):
  i = pl.program_id(0)
  zin = h_ref[...] * eps_ref[0] + a_ref[...]
  dn = (((1,), (0,)), ((), ()))
  z1 = jnp.maximum(
      lax.dot_general(zin, w1_ref[...], dn,
                      preferred_element_type=jnp.float32) + b1_ref[...], 0.0)
  z2 = jnp.maximum(
      lax.dot_general(z1, w2_ref[...], dn,
                      preferred_element_type=jnp.float32) + b2_ref[...], 0.0)
  z_ref[...] = z2
  part = jnp.concatenate([jnp.sum(z2, axis=0, keepdims=True),
                          jnp.sum(z2 * z2, axis=0, keepdims=True)], axis=0)
  @pl.when(i == 0)
  def _():
    st_ref[...] = part
  @pl.when(i != 0)
  def _():
    st_ref[...] = st_ref[...] + part


def _bn_body(z_ref, st_ref, g_ref, be_ref, out_ref, *, n_nodes, hdim):
  inv_n = 1.0 / n_nodes
  st = st_ref[...]
  mean = (st[0:1, :hdim] + st[0:1, hdim:]) * inv_n
  var = (st[1:2, :hdim] + st[1:2, hdim:]) * inv_n - mean * mean
  scale = g_ref[...] * lax.rsqrt(var + 1e-5)
  shift = be_ref[...] - mean * scale
  scale2 = jnp.concatenate([scale, scale], axis=1)
  shift2 = jnp.concatenate([shift, shift], axis=1)
  out_ref[...] = z_ref[...] * scale2 + shift2


def _head_body(b2_ref, h_ref, w1_ref, b1_ref, w2_ref, out_ref, *, hdim):
  dn = (((1,), (0,)), ((), ()))
  t = jnp.maximum(
      lax.dot_general(h_ref[...], w1_ref[...], dn,
                      preferred_element_type=jnp.float32) + b1_ref[...], 0.0)
  tw = t * w2_ref[...]
  oe = jnp.sum(tw[:, :hdim], axis=1, keepdims=True)
  oo = jnp.sum(tw[:, hdim:], axis=1, keepdims=True)
  out_ref[...] = jnp.concatenate([oe, oo], axis=1) + b2_ref[0]


def _blockdiag(w, hdim):
  z = jnp.zeros((2 * hdim, 2 * hdim), jnp.float32)
  return z.at[:hdim, :hdim].set(w).at[hdim:, hdim:].set(w)


def kernel(x, edge_index, params):
  n, hdim = x.shape
  wdim = 2 * hdim
  n2 = n // 2
  e = edge_index.shape[1]
  nb = n2 // _BLK
  assert n2 % _BLK == 0

  sc_agg = _make_sc_agg(n, e, hdim // 2)
  eflat = edge_index.reshape(2 * e)

  row_spec = pl.BlockSpec((_BLK, wdim), lambda i: (i, 0))
  w_spec = pl.BlockSpec((wdim, wdim), lambda i: (0, 0))
  b_spec = pl.BlockSpec((1, wdim), lambda i: (0, 0))
  bh_spec = pl.BlockSpec((1, hdim), lambda i: (0, 0))
  st_spec = pl.BlockSpec((2, wdim), lambda i: (0, 0))
  smem_spec = pl.BlockSpec(memory_space=pltpu.SMEM)

  mlp = pl.pallas_call(
      _mlp_body,
      grid=(nb,),
      in_specs=[smem_spec, row_spec, row_spec, w_spec, b_spec, w_spec,
                b_spec],
      out_specs=[row_spec, st_spec],
      out_shape=[jax.ShapeDtypeStruct((n2, wdim), jnp.float32),
                 jax.ShapeDtypeStruct((2, wdim), jnp.float32)],
  )

  bn = pl.pallas_call(
      functools.partial(_bn_body, n_nodes=n, hdim=hdim),
      grid=(nb,),
      in_specs=[row_spec, st_spec, bh_spec, bh_spec],
      out_specs=row_spec,
      out_shape=jax.ShapeDtypeStruct((n2, wdim), jnp.float32),
  )

  head = pl.pallas_call(
      functools.partial(_head_body, hdim=hdim),
      grid=(nb,),
      in_specs=[smem_spec, row_spec, w_spec, b_spec, b_spec],
      out_specs=pl.BlockSpec((_BLK, 2), lambda i: (i, 0)),
      out_shape=jax.ShapeDtypeStruct((n2, 2), jnp.float32),
  )

  def dup(v):
    v = v.reshape(1, hdim)
    return jnp.concatenate([v, v], axis=1)

  h128 = x.reshape(n2, wdim)
  for i in range(3):
    agg = sc_agg(h128.reshape(2 * n, hdim // 2), eflat)
    epl = (1.0 + params[f"eps{i}"]).reshape(1)
    z, st = mlp(epl, h128, agg.reshape(n2, wdim),
                _blockdiag(params[f"W1_{i}"].T, hdim),
                dup(params[f"b1_{i}"]),
                _blockdiag(params[f"W2_{i}"].T, hdim),
                dup(params[f"b2_{i}"]))
    h128 = bn(z, st, params[f"g{i}"].reshape(1, hdim),
              params[f"be{i}"].reshape(1, hdim))

  out2 = head(params["lin2_b"], h128,
              _blockdiag(params["lin1_W"].T, hdim),
              dup(params["lin1_b"]), dup(params["lin2_W"]))
  return out2.reshape(n, 1)
